# baseline (speedup 1.0000x reference)
import functools

import jax
import jax.numpy as jnp
from jax import lax
from jax.experimental import pallas as pl
from jax.experimental.pallas import tpu as pltpu
from jax.experimental.pallas import tpu_sc as plsc

NUM_TOKENS = 4096 * 200
DIM = 128

_CHUNK = 128
_CYCLE = 4
_NSLOT = 2


def _build():
    info = plsc.get_sparse_core_info()
    nw = info.num_cores * info.num_subcores
    rows_per_w = NUM_TOKENS // nw
    n_chunks = rows_per_w // _CHUNK
    n_groups = n_chunks // _CYCLE
    idx_rows_per_w = n_chunks

    mesh = plsc.VectorSubcoreMesh(core_axis_name="c", subcore_axis_name="s")

    @functools.partial(
        pl.kernel,
        mesh=mesh,
        out_type=jax.ShapeDtypeStruct((NUM_TOKENS, DIM), jnp.float32),
        scratch_types=[
            pltpu.VMEM((idx_rows_per_w, _CHUNK), jnp.int32),
            pltpu.VMEM((_CYCLE, _CHUNK, DIM), jnp.float32),
            pltpu.VMEM_SHARED((info.num_subcores, _NSLOT, _CHUNK, DIM),
                              jnp.float32),
        ] + [pltpu.SemaphoreType.DMA] * (_CYCLE + 2 * _NSLOT),
    )
    def emb(idx_hbm, table_hbm, out_hbm, idx_v, rows_v, sp, *sems):
        gsems = sems[:_CYCLE]
        csems = sems[_CYCLE:_CYCLE + _NSLOT]
        hsems = sems[_CYCLE + _NSLOT:]

        wid = lax.axis_index("s") * info.num_cores + lax.axis_index("c")
        sid = lax.axis_index("s")
        base = wid * rows_per_w

        pltpu.sync_copy(idx_hbm.at[pl.ds(wid * idx_rows_per_w, idx_rows_per_w)],
                        idx_v)

        def out_at(j):
            return out_hbm.at[pl.ds(base + j * _CHUNK, _CHUNK)]

        def gather(j, b):
            return pltpu.make_async_copy(
                table_hbm.at[idx_v.at[j]], rows_v.at[b], gsems[b])

        def xbar(b, s):
            return pltpu.make_async_copy(rows_v.at[b], sp.at[sid, s], csems[s])

        def drain_sp(j, s):
            return pltpu.make_async_copy(sp.at[sid, s], out_at(j), hsems[s])

        def do_group(j0, first):
            for k in range(_CYCLE):
                gather(j0 + k, k).start()
            for k in range(_CYCLE):
                s = k % _NSLOT
                gather(j0 + k, k).wait()
                if not first or k >= _NSLOT:
                    drain_sp(j0 + k - _NSLOT, s).wait()
                xbar(k, s).start()
                xbar(k, s).wait()
                drain_sp(j0 + k, s).start()

        do_group(0, first=True)

        def group(g, _):
            do_group(g * _CYCLE, first=False)
            return _

        lax.fori_loop(1, n_groups, group, None)

        j0 = (n_groups - 1) * _CYCLE
        drain_sp(j0 + 2, 0).wait()
        drain_sp(j0 + 3, 1).wait()

    return emb


_EMB = _build()


@jax.jit
def kernel(token_ids, weight):
    idx2d = token_ids.reshape(NUM_TOKENS // _CHUNK, _CHUNK).astype(jnp.int32)
    out = _EMB(idx2d, weight)
    return out.reshape(*token_ids.shape, DIM)


# speedup vs baseline: 1.0730x; 1.0730x over previous
import functools

import jax
import jax.numpy as jnp
from jax import lax
from jax.experimental import pallas as pl
from jax.experimental.pallas import tpu as pltpu
from jax.experimental.pallas import tpu_sc as plsc

NUM_TOKENS = 4096 * 200
DIM = 128

_CHUNK = 128
_GPB = 2
_NBUF = 2


def _build():
    info = plsc.get_sparse_core_info()
    nw = info.num_cores * info.num_subcores
    rows_per_w = NUM_TOKENS // nw
    n_chunks = rows_per_w // _CHUNK
    n_blocks = n_chunks // _GPB
    n_groups = n_blocks // _NBUF
    block_rows = _GPB * _CHUNK

    mesh = plsc.VectorSubcoreMesh(core_axis_name="c", subcore_axis_name="s")

    @functools.partial(
        pl.kernel,
        mesh=mesh,
        out_type=jax.ShapeDtypeStruct((NUM_TOKENS, DIM), jnp.float32),
        scratch_types=[
            pltpu.VMEM((n_chunks, _CHUNK), jnp.int32),
            pltpu.VMEM((_NBUF, _GPB * _CHUNK, DIM), jnp.float32),
        ] + [pltpu.SemaphoreType.DMA] * (2 * _NBUF),
    )
    def emb(idx_hbm, table_hbm, out_hbm, idx_v, rows_v, *sems):
        gsems = sems[:_NBUF]
        psems = sems[_NBUF:]

        wid = lax.axis_index("s") * info.num_cores + lax.axis_index("c")
        base = wid * rows_per_w

        pltpu.sync_copy(idx_hbm.at[pl.ds(wid * n_chunks, n_chunks)], idx_v)

        def gather(blk, b, h):
            return pltpu.make_async_copy(
                table_hbm.at[idx_v.at[blk * _GPB + h]],
                rows_v.at[b, pl.ds(h * _CHUNK, _CHUNK)],
                gsems[b])

        def put(blk, b):
            return pltpu.make_async_copy(
                rows_v.at[b],
                out_hbm.at[pl.ds(base + blk * block_rows, block_rows)],
                psems[b])

        def fill(blk, b):
            for h in range(_GPB):
                gather(blk, b, h).start()

        def flush(blk, b):
            for h in range(_GPB):
                gather(blk, b, h).wait()
            put(blk, b).start()

        for b in range(_NBUF):
            fill(b, b)

        def do_group(blk0, first, last):
            for b in range(_NBUF):
                blk = blk0 + b
                flush(blk, b)
                if not last:
                    put(blk, b).wait()
                    fill(blk + _NBUF, b)

        do_group(0, first=True, last=False)

        def group(g, _):
            do_group(g * _NBUF, first=False, last=False)
            return _

        lax.fori_loop(1, n_groups - 1, group, None)

        do_group((n_groups - 1) * _NBUF, first=False, last=True)

        blk0 = (n_groups - 1) * _NBUF
        for b in range(_NBUF):
            put(blk0 + b, b).wait()

    return emb


_EMB = _build()


@jax.jit
def kernel(token_ids, weight):
    idx2d = token_ids.reshape(NUM_TOKENS // _CHUNK, _CHUNK).astype(jnp.int32)
    out = _EMB(idx2d, weight)
    return out.reshape(*token_ids.shape, DIM)


# speedup vs baseline: 1.1421x; 1.0644x over previous
import functools

import jax
import jax.numpy as jnp
from jax import lax
from jax.experimental import pallas as pl
from jax.experimental.pallas import tpu as pltpu
from jax.experimental.pallas import tpu_sc as plsc

NUM_TOKENS = 4096 * 200
DIM = 128

_CHUNK = 80
_CYCLE = 4


def _build():
    info = plsc.get_sparse_core_info()
    nw = info.num_cores * info.num_subcores
    rows_per_w = NUM_TOKENS // nw
    n_chunks = rows_per_w // _CHUNK
    n_groups = n_chunks // _CYCLE
    idx_rows_per_w = n_chunks

    mesh = plsc.VectorSubcoreMesh(core_axis_name="c", subcore_axis_name="s")

    @functools.partial(
        pl.kernel,
        mesh=mesh,
        out_type=jax.ShapeDtypeStruct((NUM_TOKENS, DIM), jnp.float32),
        scratch_types=[
            pltpu.VMEM((idx_rows_per_w, _CHUNK), jnp.int32),
            pltpu.VMEM((_CYCLE, _CHUNK, DIM), jnp.float32),
            pltpu.VMEM_SHARED((info.num_subcores, _CYCLE, _CHUNK, DIM),
                              jnp.float32),
        ] + [pltpu.SemaphoreType.DMA] * (3 * _CYCLE),
    )
    def emb(idx_hbm, table_hbm, out_hbm, idx_v, rows_v, sp, *sems):
        gsems = sems[:_CYCLE]
        csems = sems[_CYCLE:2 * _CYCLE]
        hsems = sems[2 * _CYCLE:]

        wid = lax.axis_index("s") * info.num_cores + lax.axis_index("c")
        sid = lax.axis_index("s")
        base = wid * rows_per_w

        pltpu.sync_copy(idx_hbm.at[pl.ds(wid * idx_rows_per_w, idx_rows_per_w)],
                        idx_v)

        def out_at(j):
            return out_hbm.at[pl.ds(base + j * _CHUNK, _CHUNK)]

        def gather(j, b):
            return pltpu.make_async_copy(
                table_hbm.at[idx_v.at[j]], rows_v.at[b], gsems[b])

        def xbar(k):
            return pltpu.make_async_copy(rows_v.at[k], sp.at[sid, k], csems[k])

        def drain_sp(j, k):
            return pltpu.make_async_copy(sp.at[sid, k], out_at(j), hsems[k])

        for k in range(_CYCLE):
            gather(k, k).start()

        def do_group(j0, first, last):
            for k in range(_CYCLE):
                j = j0 + k
                gather(j, k).wait()
                if not first:
                    drain_sp(j - _CYCLE, k).wait()
                xbar(k).start()
                xbar(k).wait()
                drain_sp(j, k).start()
                if not last:
                    gather(j + _CYCLE, k).start()

        do_group(0, first=True, last=False)

        def group(g, _):
            do_group(g * _CYCLE, first=False, last=False)
            return _

        lax.fori_loop(1, n_groups - 1, group, None)

        do_group((n_groups - 1) * _CYCLE, first=False, last=True)

        j0 = (n_groups - 1) * _CYCLE
        for k in range(_CYCLE):
            drain_sp(j0 + k, k).wait()

    return emb


_EMB = _build()


@jax.jit
def kernel(token_ids, weight):
    idx2d = token_ids.reshape(NUM_TOKENS // _CHUNK, _CHUNK).astype(jnp.int32)
    out = _EMB(idx2d, weight)
    return out.reshape(*token_ids.shape, DIM)
